# baseline (speedup 1.0000x reference)
import functools

import jax
import jax.numpy as jnp
from jax import lax
from jax.experimental import pallas as pl
from jax.experimental.pallas import tpu as pltpu
from jax.experimental.pallas import tpu_sc as plsc

NC = 2
NS = 16
NW = NC * NS
LANES = 16

ROWS = 8
C = ROWS * 128
GROUPS = C // LANES

F32 = jnp.float32
I32 = jnp.int32

S3 = 3.0 ** 0.5
S5 = 5.0 ** 0.5
S15 = 15.0 ** 0.5
PI = 3.14159265358979
LOGEPS = -27.631021
LN2 = 0.6931471805599453
SQRT2 = 1.4142135


def _edge_kernel_body(nchunk, base_cnt, extra_w,
                      tbl_hbm, es_hbm, ed_hbm,
                      sh_hbm, len_hbm, cut_hbm, log_hbm,
                      idx_s, idx_d, rows_s, rows_d,
                      shb, lb, cb, gb, sem_s, sem_d):
    wid = lax.axis_index("s") * NC + lax.axis_index("c")
    my_cnt = base_cnt + jnp.where(wid < extra_w, 1, 0).astype(I32)

    iota = lax.iota(I32, 16)
    ones = jnp.full((16,), 1.0, F32)
    cols = [jnp.full((16,), c, I32) for c in range(6)]

    def chunk_body(i, carry):
        c = wid + i * NW
        rbase = pl.multiple_of(c * ROWS, 8)
        ebase = pl.multiple_of(c * C, 128)
        pltpu.sync_copy(es_hbm.at[pl.ds(rbase, ROWS)], idx_s)
        pltpu.sync_copy(ed_hbm.at[pl.ds(rbase, ROWS)], idx_d)
        handles = []
        for j in range(ROWS):
            handles.append(pltpu.async_copy(
                tbl_hbm.at[idx_s.at[j]], rows_s.at[pl.ds(j * 128, 128)], sem_s))
            handles.append(pltpu.async_copy(
                tbl_hbm.at[idx_d.at[j]], rows_d.at[pl.ds(j * 128, 128)], sem_d))
        for h in handles:
            h.wait()

        def group(g, gcarry):
            row = iota + g * LANES
            sx = plsc.load_gather(rows_s, [row, cols[0]])
            sy = plsc.load_gather(rows_s, [row, cols[1]])
            sz = plsc.load_gather(rows_s, [row, cols[2]])
            dx = plsc.load_gather(rows_d, [row, cols[3]])
            dy = plsc.load_gather(rows_d, [row, cols[4]])
            dz = plsc.load_gather(rows_d, [row, cols[5]])
            vx = sx - dx
            vy = sy - dy
            vz = sz - dz
            r2 = vx * vx + vy * vy + vz * vz
            bi = lax.bitcast_convert_type(r2, I32)
            y = lax.bitcast_convert_type(
                jnp.int32(0x5F3759DF) - lax.shift_right_logical(bi, 1), F32)
            y = y * (1.5 - 0.5 * r2 * y * y)
            y = y * (1.5 - 0.5 * r2 * y * y)
            y = y * (1.5 - 0.5 * r2 * y * y)
            L = r2 * y
            ux = vx * y
            uy = vy * y
            uz = vz * y
            xx = ux * ux
            yy = uy * uy
            zz = uz * uz
            zarg = (0.5 * PI) * (5.0 - L)
            z2 = zarg * zarg
            p = -1.0 / 39916800.0
            p = p * z2 + 1.0 / 362880.0
            p = p * z2 - 1.0 / 5040.0
            p = p * z2 + 1.0 / 120.0
            p = p * z2 - 1.0 / 6.0
            p = p * z2 + 1.0
            h = zarg * p
            decay = h * h
            lt4 = L < 4.0
            gt5 = L > 5.0
            cut = jnp.where(lt4, 1.0, jnp.where(gt5, 0.0, decay))
            cut = jnp.maximum(cut, 1e-12)
            ib = lax.bitcast_convert_type(cut, I32)
            e = lax.shift_right_logical(ib, 23) - 127
            m = lax.bitcast_convert_type(
                (ib & jnp.int32(0x007FFFFF)) | jnp.int32(0x3F800000), F32)
            big = m > SQRT2
            m = jnp.where(big, 0.5 * m, m)
            ef = (e + jnp.where(big, 1, 0)).astype(F32)
            w = (m - 1.0) / (m + 1.0)
            w2 = w * w
            q = 2.0 / 9.0
            q = q * w2 + 2.0 / 7.0
            q = q * w2 + 2.0 / 5.0
            q = q * w2 + 2.0 / 3.0
            q = q * w2 + 2.0
            lg = ef * LN2 + w * q
            lg = jnp.where(lt4, 0.0, jnp.where(gt5, LOGEPS, lg))
            gs = pl.ds(g * LANES, LANES)
            shb[0, gs] = ones
            shb[1, gs] = S3 * ux
            shb[2, gs] = S3 * uy
            shb[3, gs] = S3 * uz
            shb[4, gs] = S15 * ux * uz
            shb[5, gs] = S15 * ux * uy
            shb[6, gs] = S5 * (yy - 0.5 * (xx + zz))
            shb[7, gs] = S15 * uy * uz
            shb[8, gs] = (0.5 * S15) * (zz - xx)
            lb[gs] = L
            cb[gs] = cut
            gb[gs] = lg
            return gcarry

        lax.fori_loop(0, GROUPS, group, 0)
        for k in range(9):
            pltpu.sync_copy(shb.at[k], sh_hbm.at[k, pl.ds(ebase, C)])
        pltpu.sync_copy(lb, len_hbm.at[pl.ds(ebase, C)])
        pltpu.sync_copy(cb, cut_hbm.at[pl.ds(ebase, C)])
        pltpu.sync_copy(gb, log_hbm.at[pl.ds(ebase, C)])
        return carry

    lax.fori_loop(0, my_cnt, chunk_body, 0)


def kernel(x_src, x_dst, edge_src, edge_dst):
    n = x_src.shape[0]
    e = edge_src.shape[0]
    assert e % 128 == 0
    nrows = e // 128
    nchunk = nrows // ROWS
    assert nchunk * ROWS == nrows
    base_cnt = nchunk // NW
    extra_w = nchunk - base_cnt * NW

    pblk = 1000
    assert n % pblk == 0

    def _pad_body(xs_ref, xd_ref, out_ref):
        out_ref[...] = jnp.concatenate(
            [xs_ref[...], xd_ref[...], jnp.zeros((pblk, 2), F32)], axis=1)

    tbl = pl.pallas_call(
        _pad_body,
        grid=(n // pblk,),
        in_specs=[
            pl.BlockSpec((pblk, 3), lambda i: (i, 0)),
            pl.BlockSpec((pblk, 3), lambda i: (i, 0)),
        ],
        out_specs=pl.BlockSpec((pblk, 8), lambda i: (i, 0)),
        out_shape=jax.ShapeDtypeStruct((n, 8), F32),
    )(x_src, x_dst)

    es2 = edge_src.reshape(nrows, 128)
    ed2 = edge_dst.reshape(nrows, 128)

    mesh = plsc.VectorSubcoreMesh(core_axis_name="c", subcore_axis_name="s")
    run = pl.kernel(
        functools.partial(_edge_kernel_body, nchunk, base_cnt, extra_w),
        out_type=(
            jax.ShapeDtypeStruct((9, e), F32),
            jax.ShapeDtypeStruct((e,), F32),
            jax.ShapeDtypeStruct((e,), F32),
            jax.ShapeDtypeStruct((e,), F32),
        ),
        mesh=mesh,
        compiler_params=pltpu.CompilerParams(
            needs_layout_passes=False, use_tc_tiling_on_sc=False),
        scratch_types=[
            pltpu.VMEM((ROWS, 128), I32),
            pltpu.VMEM((ROWS, 128), I32),
            pltpu.VMEM((C, 8), F32),
            pltpu.VMEM((C, 8), F32),
            pltpu.VMEM((9, C), F32),
            pltpu.VMEM((C,), F32),
            pltpu.VMEM((C,), F32),
            pltpu.VMEM((C,), F32),
            pltpu.SemaphoreType.DMA,
            pltpu.SemaphoreType.DMA,
        ],
    )
    sh, length, cut, lg = run(tbl, es2, ed2)
    return (edge_src, edge_dst, sh.T, length, cut, lg)


# speedup vs baseline: 3.2893x; 3.2893x over previous
import functools

import jax
import jax.numpy as jnp
from jax import lax
from jax.experimental import pallas as pl
from jax.experimental.pallas import tpu as pltpu
from jax.experimental.pallas import tpu_sc as plsc

NC = 2
NS = 16
NW = NC * NS
LANES = 16

ROWS = 8
C = ROWS * 128
GROUPS = C // LANES

F32 = jnp.float32
I32 = jnp.int32

S3 = 3.0 ** 0.5
S5 = 5.0 ** 0.5
S15 = 15.0 ** 0.5
PI = 3.14159265358979
LOGEPS = -27.631021
LN2 = 0.6931471805599453
SQRT2 = 1.4142135


def _edge_kernel_body(nchunk, base_cnt, extra_w,
                      tbl_hbm, es_hbm, ed_hbm,
                      sh_hbm, len_hbm, cut_hbm, log_hbm,
                      idx_s, idx_d, rows_s, rows_d,
                      shb, lb, cb, gb, sem_s, sem_d):
    wid = lax.axis_index("s") * NC + lax.axis_index("c")
    my_cnt = base_cnt + jnp.where(wid < extra_w, 1, 0).astype(I32)

    iota = lax.iota(I32, 16)
    ones = jnp.full((16,), 1.0, F32)
    cols = [jnp.full((16,), c, I32) for c in range(6)]

    def chunk_body(i, carry):
        c = wid + i * NW
        rbase = pl.multiple_of(c * ROWS, 8)
        ebase = pl.multiple_of(c * C, 128)
        pltpu.sync_copy(es_hbm.at[pl.ds(rbase, ROWS)], idx_s)
        pltpu.sync_copy(ed_hbm.at[pl.ds(rbase, ROWS)], idx_d)
        handles = []
        for j in range(ROWS):
            handles.append(pltpu.async_copy(
                tbl_hbm.at[idx_s.at[j]], rows_s.at[pl.ds(j * 128, 128)], sem_s))
            handles.append(pltpu.async_copy(
                tbl_hbm.at[idx_d.at[j]], rows_d.at[pl.ds(j * 128, 128)], sem_d))
        for h in handles:
            h.wait()

        def group(g, gcarry):
            row = iota + g * LANES
            sx = plsc.load_gather(rows_s, [row, cols[0]])
            sy = plsc.load_gather(rows_s, [row, cols[1]])
            sz = plsc.load_gather(rows_s, [row, cols[2]])
            dx = plsc.load_gather(rows_d, [row, cols[3]])
            dy = plsc.load_gather(rows_d, [row, cols[4]])
            dz = plsc.load_gather(rows_d, [row, cols[5]])
            vx = sx - dx
            vy = sy - dy
            vz = sz - dz
            r2 = vx * vx + vy * vy + vz * vz
            bi = lax.bitcast_convert_type(r2, I32)
            y = lax.bitcast_convert_type(
                jnp.int32(0x5F3759DF) - lax.shift_right_logical(bi, 1), F32)
            y = y * (1.5 - 0.5 * r2 * y * y)
            y = y * (1.5 - 0.5 * r2 * y * y)
            y = y * (1.5 - 0.5 * r2 * y * y)
            L = r2 * y
            ux = vx * y
            uy = vy * y
            uz = vz * y
            xx = ux * ux
            yy = uy * uy
            zz = uz * uz
            zarg = (0.5 * PI) * (5.0 - L)
            z2 = zarg * zarg
            p = -1.0 / 39916800.0
            p = p * z2 + 1.0 / 362880.0
            p = p * z2 - 1.0 / 5040.0
            p = p * z2 + 1.0 / 120.0
            p = p * z2 - 1.0 / 6.0
            p = p * z2 + 1.0
            h = zarg * p
            decay = h * h
            lt4 = L < 4.0
            gt5 = L > 5.0
            cut = jnp.where(lt4, 1.0, jnp.where(gt5, 0.0, decay))
            cut = jnp.maximum(cut, 1e-12)
            ib = lax.bitcast_convert_type(cut, I32)
            e = lax.shift_right_logical(ib, 23) - 127
            m = lax.bitcast_convert_type(
                (ib & jnp.int32(0x007FFFFF)) | jnp.int32(0x3F800000), F32)
            big = m > SQRT2
            m = jnp.where(big, 0.5 * m, m)
            ef = (e + jnp.where(big, 1, 0)).astype(F32)
            w = (m - 1.0) / (m + 1.0)
            w2 = w * w
            q = 2.0 / 9.0
            q = q * w2 + 2.0 / 7.0
            q = q * w2 + 2.0 / 5.0
            q = q * w2 + 2.0 / 3.0
            q = q * w2 + 2.0
            lg = ef * LN2 + w * q
            lg = jnp.where(lt4, 0.0, jnp.where(gt5, LOGEPS, lg))
            gs = pl.ds(g * LANES, LANES)
            jl = g // 8
            ls = pl.ds((g % 8) * LANES, LANES)
            shb[0, jl, 0, ls] = ones
            shb[0, jl, 1, ls] = S3 * ux
            shb[0, jl, 2, ls] = S3 * uy
            shb[0, jl, 3, ls] = S3 * uz
            shb[0, jl, 4, ls] = S15 * ux * uz
            shb[0, jl, 5, ls] = S15 * ux * uy
            shb[0, jl, 6, ls] = S5 * (yy - 0.5 * (xx + zz))
            shb[0, jl, 7, ls] = S15 * uy * uz
            shb[1, jl, 0, ls] = (0.5 * S15) * (zz - xx)
            lb[gs] = L
            cb[gs] = cut
            gb[gs] = lg
            return gcarry

        lax.fori_loop(0, GROUPS, group, 0)
        j0 = pl.multiple_of(c * (C // 128), 8)
        pltpu.sync_copy(shb, sh_hbm.at[:, pl.ds(j0, C // 128)])
        pltpu.sync_copy(lb, len_hbm.at[pl.ds(ebase, C)])
        pltpu.sync_copy(cb, cut_hbm.at[pl.ds(ebase, C)])
        pltpu.sync_copy(gb, log_hbm.at[pl.ds(ebase, C)])
        return carry

    lax.fori_loop(0, my_cnt, chunk_body, 0)


def kernel(x_src, x_dst, edge_src, edge_dst):
    n = x_src.shape[0]
    e = edge_src.shape[0]
    assert e % 128 == 0
    nrows = e // 128
    nchunk = nrows // ROWS
    assert nchunk * ROWS == nrows
    base_cnt = nchunk // NW
    extra_w = nchunk - base_cnt * NW

    pblk = 1000
    assert n % pblk == 0

    def _pad_body(xs_ref, xd_ref, out_ref):
        out_ref[...] = jnp.concatenate(
            [xs_ref[...], xd_ref[...], jnp.zeros((pblk, 2), F32)], axis=1)

    tbl = pl.pallas_call(
        _pad_body,
        grid=(n // pblk,),
        in_specs=[
            pl.BlockSpec((pblk, 3), lambda i: (i, 0)),
            pl.BlockSpec((pblk, 3), lambda i: (i, 0)),
        ],
        out_specs=pl.BlockSpec((pblk, 8), lambda i: (i, 0)),
        out_shape=jax.ShapeDtypeStruct((n, 8), F32),
    )(x_src, x_dst)

    es2 = edge_src.reshape(nrows, 128)
    ed2 = edge_dst.reshape(nrows, 128)

    mesh = plsc.VectorSubcoreMesh(core_axis_name="c", subcore_axis_name="s")
    run = pl.kernel(
        functools.partial(_edge_kernel_body, nchunk, base_cnt, extra_w),
        out_type=(
            jax.ShapeDtypeStruct((2, e // 128, 8, 128), F32),
            jax.ShapeDtypeStruct((e,), F32),
            jax.ShapeDtypeStruct((e,), F32),
            jax.ShapeDtypeStruct((e,), F32),
        ),
        mesh=mesh,
        compiler_params=pltpu.CompilerParams(
            needs_layout_passes=False, use_tc_tiling_on_sc=False),
        scratch_types=[
            pltpu.VMEM((ROWS, 128), I32),
            pltpu.VMEM((ROWS, 128), I32),
            pltpu.VMEM((C, 8), F32),
            pltpu.VMEM((C, 8), F32),
            pltpu.VMEM((2, C // 128, 8, 128), F32),
            pltpu.VMEM((C,), F32),
            pltpu.VMEM((C,), F32),
            pltpu.VMEM((C,), F32),
            pltpu.SemaphoreType.DMA,
            pltpu.SemaphoreType.DMA,
        ],
    )
    sh4, length, cut, lg = run(tbl, es2, ed2)
    sh = sh4.transpose(1, 3, 0, 2).reshape(e, 16)[:, :9]
    return (edge_src, edge_dst, sh, length, cut, lg)


# speedup vs baseline: 4.4786x; 1.3616x over previous
import functools

import jax
import jax.numpy as jnp
from jax import lax
from jax.experimental import pallas as pl
from jax.experimental.pallas import tpu as pltpu
from jax.experimental.pallas import tpu_sc as plsc

NC = 2
NS = 16
NW = NC * NS
LANES = 16

ROWS = 8
C = ROWS * 128
GROUPS = C // LANES

F32 = jnp.float32
I32 = jnp.int32

S3 = 3.0 ** 0.5
S5 = 5.0 ** 0.5
S15 = 15.0 ** 0.5
PI = 3.14159265358979
LOGEPS = -27.631021
LN2 = 0.6931471805599453
SQRT2 = 1.4142135


def _edge_kernel_body(base_cnt, extra_w,
                      tbl_hbm, es_hbm, ed_hbm,
                      sh_hbm, len_hbm, cut_hbm, log_hbm,
                      idx_s, idx_d, rows_s, rows_d,
                      shb, lb, cb, gb,
                      sem_gs0, sem_gs1, sem_gd0, sem_gd1, sem_wb0, sem_wb1):
    wid = lax.axis_index("s") * NC + lax.axis_index("c")
    my_cnt = base_cnt + jnp.where(wid < extra_w, 1, 0).astype(I32)
    sem_gs = [sem_gs0, sem_gs1]
    sem_gd = [sem_gd0, sem_gd1]
    sem_wb = [sem_wb0, sem_wb1]

    iota = lax.iota(I32, 16)
    ones = jnp.full((16,), 1.0, F32)
    cols = [jnp.full((16,), c, I32) for c in range(6)]

    def stage_and_fire(cc, q):
        rb = pl.multiple_of(cc * ROWS, 8)
        pltpu.sync_copy(es_hbm.at[pl.ds(rb, ROWS)], idx_s.at[q])
        pltpu.sync_copy(ed_hbm.at[pl.ds(rb, ROWS)], idx_d.at[q])
        for j in range(ROWS):
            pltpu.async_copy(tbl_hbm.at[idx_s.at[q, j]],
                             rows_s.at[q, pl.ds(j * 128, 128)], sem_gs[q])
            pltpu.async_copy(tbl_hbm.at[idx_d.at[q, j]],
                             rows_d.at[q, pl.ds(j * 128, 128)], sem_gd[q])

    def wait_gathers(q):
        for j in range(ROWS):
            pltpu.make_async_copy(tbl_hbm.at[idx_s.at[q, j]],
                                  rows_s.at[q, pl.ds(j * 128, 128)],
                                  sem_gs[q]).wait()
            pltpu.make_async_copy(tbl_hbm.at[idx_d.at[q, j]],
                                  rows_d.at[q, pl.ds(j * 128, 128)],
                                  sem_gd[q]).wait()

    def wb_descs(cc, q):
        j0 = pl.multiple_of(cc * (C // 128), 8)
        eb = pl.multiple_of(cc * C, 128)
        return [
            (shb.at[q], sh_hbm.at[:, pl.ds(j0, C // 128)]),
            (lb.at[q], len_hbm.at[pl.ds(eb, C)]),
            (cb.at[q], cut_hbm.at[pl.ds(eb, C)]),
            (gb.at[q], log_hbm.at[pl.ds(eb, C)]),
        ]

    def compute(q):
        rs = rows_s.at[q]
        rd = rows_d.at[q]

        def group(g, gcarry):
            row = iota + g * LANES
            sx = plsc.load_gather(rs, [row, cols[0]])
            sy = plsc.load_gather(rs, [row, cols[1]])
            sz = plsc.load_gather(rs, [row, cols[2]])
            dx = plsc.load_gather(rd, [row, cols[3]])
            dy = plsc.load_gather(rd, [row, cols[4]])
            dz = plsc.load_gather(rd, [row, cols[5]])
            vx = sx - dx
            vy = sy - dy
            vz = sz - dz
            r2 = vx * vx + vy * vy + vz * vz
            bi = lax.bitcast_convert_type(r2, I32)
            y = lax.bitcast_convert_type(
                jnp.int32(0x5F3759DF) - lax.shift_right_logical(bi, 1), F32)
            y = y * (1.5 - 0.5 * r2 * y * y)
            y = y * (1.5 - 0.5 * r2 * y * y)
            y = y * (1.5 - 0.5 * r2 * y * y)
            L = r2 * y
            ux = vx * y
            uy = vy * y
            uz = vz * y
            xx = ux * ux
            yy = uy * uy
            zz = uz * uz
            zarg = (0.5 * PI) * (5.0 - L)
            z2 = zarg * zarg
            p = -1.0 / 39916800.0
            p = p * z2 + 1.0 / 362880.0
            p = p * z2 - 1.0 / 5040.0
            p = p * z2 + 1.0 / 120.0
            p = p * z2 - 1.0 / 6.0
            p = p * z2 + 1.0
            h = zarg * p
            decay = h * h
            lt4 = L < 4.0
            gt5 = L > 5.0
            cut = jnp.where(lt4, 1.0, jnp.where(gt5, 0.0, decay))
            cut = jnp.maximum(cut, 1e-12)
            ib = lax.bitcast_convert_type(cut, I32)
            e = lax.shift_right_logical(ib, 23) - 127
            m = lax.bitcast_convert_type(
                (ib & jnp.int32(0x007FFFFF)) | jnp.int32(0x3F800000), F32)
            big = m > SQRT2
            m = jnp.where(big, 0.5 * m, m)
            ef = (e + jnp.where(big, 1, 0)).astype(F32)
            w = (m - 1.0) / (m + 1.0)
            w2 = w * w
            q_ = 2.0 / 9.0
            q_ = q_ * w2 + 2.0 / 7.0
            q_ = q_ * w2 + 2.0 / 5.0
            q_ = q_ * w2 + 2.0 / 3.0
            q_ = q_ * w2 + 2.0
            lg = ef * LN2 + w * q_
            lg = jnp.where(lt4, 0.0, jnp.where(gt5, LOGEPS, lg))
            gs = pl.ds(g * LANES, LANES)
            jl = g // 8
            ls = pl.ds((g % 8) * LANES, LANES)
            shb[q, 0, jl, 0, ls] = ones
            shb[q, 0, jl, 1, ls] = S3 * ux
            shb[q, 0, jl, 2, ls] = S3 * uy
            shb[q, 0, jl, 3, ls] = S3 * uz
            shb[q, 0, jl, 4, ls] = S15 * ux * uz
            shb[q, 0, jl, 5, ls] = S15 * ux * uy
            shb[q, 0, jl, 6, ls] = S5 * (yy - 0.5 * (xx + zz))
            shb[q, 0, jl, 7, ls] = S15 * uy * uz
            shb[q, 1, jl, 0, ls] = (0.5 * S15) * (zz - xx)
            lb[q, gs] = L
            cb[q, gs] = cut
            gb[q, gs] = lg
            return gcarry

        lax.fori_loop(0, GROUPS, group, 0)

    def do_chunk(i, c, q):
        @pl.when(i + 1 < my_cnt)
        def _prefetch():
            stage_and_fire(c + NW, 1 - q)

        wait_gathers(q)

        @pl.when(i >= 2)
        def _drain_wb():
            for s_ref, d_ref in wb_descs(c, q):
                pltpu.make_async_copy(s_ref, d_ref, sem_wb[q]).wait()

        compute(q)
        for s_ref, d_ref in wb_descs(c, q):
            pltpu.async_copy(s_ref, d_ref, sem_wb[q])

    stage_and_fire(wid, 0)

    def pair_body(k, carry):
        i0 = 2 * k
        c0 = wid + i0 * NW
        do_chunk(i0, c0, 0)

        @pl.when(i0 + 1 < my_cnt)
        def _odd():
            do_chunk(i0 + 1, c0 + NW, 1)

        return carry

    lax.fori_loop(0, (my_cnt + 1) // 2, pair_body, 0)

    for q in (0, 1):
        for s_ref, d_ref in wb_descs(wid, q):
            pltpu.make_async_copy(s_ref, d_ref, sem_wb[q]).wait()


def kernel(x_src, x_dst, edge_src, edge_dst):
    n = x_src.shape[0]
    e = edge_src.shape[0]
    assert e % 128 == 0
    nrows = e // 128
    nchunk = nrows // ROWS
    assert nchunk * ROWS == nrows
    base_cnt = nchunk // NW
    extra_w = nchunk - base_cnt * NW

    pblk = 1000
    assert n % pblk == 0

    def _pad_body(xs_ref, xd_ref, out_ref):
        out_ref[...] = jnp.concatenate(
            [xs_ref[...], xd_ref[...], jnp.zeros((pblk, 2), F32)], axis=1)

    tbl = pl.pallas_call(
        _pad_body,
        grid=(n // pblk,),
        in_specs=[
            pl.BlockSpec((pblk, 3), lambda i: (i, 0)),
            pl.BlockSpec((pblk, 3), lambda i: (i, 0)),
        ],
        out_specs=pl.BlockSpec((pblk, 8), lambda i: (i, 0)),
        out_shape=jax.ShapeDtypeStruct((n, 8), F32),
    )(x_src, x_dst)

    es2 = edge_src.reshape(nrows, 128)
    ed2 = edge_dst.reshape(nrows, 128)

    mesh = plsc.VectorSubcoreMesh(core_axis_name="c", subcore_axis_name="s")
    run = pl.kernel(
        functools.partial(_edge_kernel_body, base_cnt, extra_w),
        out_type=(
            jax.ShapeDtypeStruct((2, e // 128, 8, 128), F32),
            jax.ShapeDtypeStruct((e,), F32),
            jax.ShapeDtypeStruct((e,), F32),
            jax.ShapeDtypeStruct((e,), F32),
        ),
        mesh=mesh,
        compiler_params=pltpu.CompilerParams(
            needs_layout_passes=False, use_tc_tiling_on_sc=False),
        scratch_types=[
            pltpu.VMEM((2, ROWS, 128), I32),
            pltpu.VMEM((2, ROWS, 128), I32),
            pltpu.VMEM((2, C, 8), F32),
            pltpu.VMEM((2, C, 8), F32),
            pltpu.VMEM((2, 2, C // 128, 8, 128), F32),
            pltpu.VMEM((2, C), F32),
            pltpu.VMEM((2, C), F32),
            pltpu.VMEM((2, C), F32),
            pltpu.SemaphoreType.DMA,
            pltpu.SemaphoreType.DMA,
            pltpu.SemaphoreType.DMA,
            pltpu.SemaphoreType.DMA,
            pltpu.SemaphoreType.DMA,
            pltpu.SemaphoreType.DMA,
        ],
    )
    sh4, length, cut, lg = run(tbl, es2, ed2)
    sh = sh4.transpose(1, 3, 0, 2).reshape(e, 16)[:, :9]
    return (edge_src, edge_dst, sh, length, cut, lg)


# speedup vs baseline: 4.6824x; 1.0455x over previous
import functools

import jax
import jax.numpy as jnp
from jax import lax
from jax.experimental import pallas as pl
from jax.experimental.pallas import tpu as pltpu
from jax.experimental.pallas import tpu_sc as plsc

NC = 2
NS = 16
NW = NC * NS
LANES = 16

ROWS = 8
C = ROWS * 128
GROUPS = C // LANES

F32 = jnp.float32
I32 = jnp.int32

S3 = 3.0 ** 0.5
S5 = 5.0 ** 0.5
S15 = 15.0 ** 0.5
PI = 3.14159265358979
LOGEPS = -27.631021
LN2 = 0.6931471805599453
SQRT2 = 1.4142135


def _edge_kernel_body(base_cnt, extra_w,
                      tbl_hbm, es_hbm, ed_hbm,
                      sh_hbm, len_hbm, cut_hbm, log_hbm,
                      idx_s, idx_d, rows_s, rows_d,
                      shb, lb, cb, gb,
                      sem_gs0, sem_gs1, sem_gd0, sem_gd1, sem_wb0, sem_wb1):
    wid = lax.axis_index("s") * NC + lax.axis_index("c")
    my_cnt = base_cnt + jnp.where(wid < extra_w, 1, 0).astype(I32)
    sem_gs = [sem_gs0, sem_gs1]
    sem_gd = [sem_gd0, sem_gd1]
    sem_wb = [sem_wb0, sem_wb1]

    iota = lax.iota(I32, 16)
    ones = jnp.full((16,), 1.0, F32)
    cols = [jnp.full((16,), c, I32) for c in range(6)]

    def stage_and_fire(cc, q):
        rb = pl.multiple_of(cc * ROWS, 8)
        pltpu.sync_copy(es_hbm.at[pl.ds(rb, ROWS)], idx_s.at[q])
        pltpu.sync_copy(ed_hbm.at[pl.ds(rb, ROWS)], idx_d.at[q])
        for j in range(ROWS):
            pltpu.async_copy(tbl_hbm.at[idx_s.at[q, j]],
                             rows_s.at[q, pl.ds(j * 128, 128)], sem_gs[q])
            pltpu.async_copy(tbl_hbm.at[idx_d.at[q, j]],
                             rows_d.at[q, pl.ds(j * 128, 128)], sem_gd[q])

    def wait_gathers(q):
        for j in range(ROWS):
            pltpu.make_async_copy(tbl_hbm.at[idx_s.at[q, j]],
                                  rows_s.at[q, pl.ds(j * 128, 128)],
                                  sem_gs[q]).wait()
            pltpu.make_async_copy(tbl_hbm.at[idx_d.at[q, j]],
                                  rows_d.at[q, pl.ds(j * 128, 128)],
                                  sem_gd[q]).wait()

    def wb_descs(cc, q):
        j0 = pl.multiple_of(cc * (C // 128), 8)
        eb = pl.multiple_of(cc * C, 128)
        return [
            (shb.at[q], sh_hbm.at[:, pl.ds(j0, C // 128)]),
            (lb.at[q], len_hbm.at[pl.ds(eb, C)]),
            (cb.at[q], cut_hbm.at[pl.ds(eb, C)]),
            (gb.at[q], log_hbm.at[pl.ds(eb, C)]),
        ]

    def compute(q):
        rs = rows_s.at[q]
        rd = rows_d.at[q]

        def group(g, gcarry):
            row = iota + g * LANES
            sx = plsc.load_gather(rs, [row, cols[0]])
            sy = plsc.load_gather(rs, [row, cols[1]])
            sz = plsc.load_gather(rs, [row, cols[2]])
            dx = plsc.load_gather(rd, [row, cols[3]])
            dy = plsc.load_gather(rd, [row, cols[4]])
            dz = plsc.load_gather(rd, [row, cols[5]])
            vx = sx - dx
            vy = sy - dy
            vz = sz - dz
            r2 = vx * vx + vy * vy + vz * vz
            bi = lax.bitcast_convert_type(r2, I32)
            y = lax.bitcast_convert_type(
                jnp.int32(0x5F3759DF) - lax.shift_right_logical(bi, 1), F32)
            y = y * (1.5 - 0.5 * r2 * y * y)
            y = y * (1.5 - 0.5 * r2 * y * y)
            y = y * (1.5 - 0.5 * r2 * y * y)
            L = r2 * y
            ux = vx * y
            uy = vy * y
            uz = vz * y
            xx = ux * ux
            yy = uy * uy
            zz = uz * uz
            zarg = (0.5 * PI) * (5.0 - L)
            z2 = zarg * zarg
            p = 1.0 / 362880.0
            p = p * z2 - 1.0 / 5040.0
            p = p * z2 + 1.0 / 120.0
            p = p * z2 - 1.0 / 6.0
            p = p * z2 + 1.0
            h = zarg * p
            decay = h * h
            lt4 = L < 4.0
            gt5 = L > 5.0
            cut = jnp.where(lt4, 1.0, jnp.where(gt5, 0.0, decay))
            cut = jnp.maximum(cut, 1e-12)
            ib = lax.bitcast_convert_type(cut, I32)
            ef = (lax.shift_right_logical(ib, 23) - 127).astype(F32)
            m = lax.bitcast_convert_type(
                (ib & jnp.int32(0x007FFFFF)) | jnp.int32(0x3F800000), F32)
            w = (m - 1.0) / (m + 1.0)
            w2 = w * w
            q_ = 2.0 / 7.0
            q_ = q_ * w2 + 2.0 / 5.0
            q_ = q_ * w2 + 2.0 / 3.0
            q_ = q_ * w2 + 2.0
            lg = ef * LN2 + w * q_
            lg = jnp.where(lt4, 0.0, jnp.where(gt5, LOGEPS, lg))
            gs = pl.ds(g * LANES, LANES)
            jl = g // 8
            ls = pl.ds((g % 8) * LANES, LANES)
            shb[q, 0, jl, 0, ls] = ones
            shb[q, 0, jl, 1, ls] = S3 * ux
            shb[q, 0, jl, 2, ls] = S3 * uy
            shb[q, 0, jl, 3, ls] = S3 * uz
            shb[q, 0, jl, 4, ls] = S15 * ux * uz
            shb[q, 0, jl, 5, ls] = S15 * ux * uy
            shb[q, 0, jl, 6, ls] = S5 * (yy - 0.5 * (xx + zz))
            shb[q, 0, jl, 7, ls] = S15 * uy * uz
            shb[q, 1, jl, 0, ls] = (0.5 * S15) * (zz - xx)
            lb[q, gs] = L
            cb[q, gs] = cut
            gb[q, gs] = lg
            return gcarry

        lax.fori_loop(0, GROUPS, group, 0, unroll=4)

    def do_chunk(i, c, q):
        @pl.when(i + 1 < my_cnt)
        def _prefetch():
            stage_and_fire(c + NW, 1 - q)

        wait_gathers(q)

        @pl.when(i >= 2)
        def _drain_wb():
            for s_ref, d_ref in wb_descs(c, q):
                pltpu.make_async_copy(s_ref, d_ref, sem_wb[q]).wait()

        compute(q)
        for s_ref, d_ref in wb_descs(c, q):
            pltpu.async_copy(s_ref, d_ref, sem_wb[q])

    stage_and_fire(wid, 0)

    def pair_body(k, carry):
        i0 = 2 * k
        c0 = wid + i0 * NW
        do_chunk(i0, c0, 0)

        @pl.when(i0 + 1 < my_cnt)
        def _odd():
            do_chunk(i0 + 1, c0 + NW, 1)

        return carry

    lax.fori_loop(0, (my_cnt + 1) // 2, pair_body, 0)

    for q in (0, 1):
        for s_ref, d_ref in wb_descs(wid, q):
            pltpu.make_async_copy(s_ref, d_ref, sem_wb[q]).wait()


def kernel(x_src, x_dst, edge_src, edge_dst):
    n = x_src.shape[0]
    e = edge_src.shape[0]
    assert e % 128 == 0
    nrows = e // 128
    nchunk = nrows // ROWS
    assert nchunk * ROWS == nrows
    base_cnt = nchunk // NW
    extra_w = nchunk - base_cnt * NW

    pblk = 1000
    assert n % pblk == 0

    def _pad_body(xs_ref, xd_ref, out_ref):
        out_ref[...] = jnp.concatenate(
            [xs_ref[...], xd_ref[...], jnp.zeros((pblk, 2), F32)], axis=1)

    tbl = pl.pallas_call(
        _pad_body,
        grid=(n // pblk,),
        in_specs=[
            pl.BlockSpec((pblk, 3), lambda i: (i, 0)),
            pl.BlockSpec((pblk, 3), lambda i: (i, 0)),
        ],
        out_specs=pl.BlockSpec((pblk, 8), lambda i: (i, 0)),
        out_shape=jax.ShapeDtypeStruct((n, 8), F32),
    )(x_src, x_dst)

    es2 = edge_src.reshape(nrows, 128)
    ed2 = edge_dst.reshape(nrows, 128)

    mesh = plsc.VectorSubcoreMesh(core_axis_name="c", subcore_axis_name="s")
    run = pl.kernel(
        functools.partial(_edge_kernel_body, base_cnt, extra_w),
        out_type=(
            jax.ShapeDtypeStruct((2, e // 128, 8, 128), F32),
            jax.ShapeDtypeStruct((e,), F32),
            jax.ShapeDtypeStruct((e,), F32),
            jax.ShapeDtypeStruct((e,), F32),
        ),
        mesh=mesh,
        compiler_params=pltpu.CompilerParams(
            needs_layout_passes=False, use_tc_tiling_on_sc=False),
        scratch_types=[
            pltpu.VMEM((2, ROWS, 128), I32),
            pltpu.VMEM((2, ROWS, 128), I32),
            pltpu.VMEM((2, C, 8), F32),
            pltpu.VMEM((2, C, 8), F32),
            pltpu.VMEM((2, 2, C // 128, 8, 128), F32),
            pltpu.VMEM((2, C), F32),
            pltpu.VMEM((2, C), F32),
            pltpu.VMEM((2, C), F32),
            pltpu.SemaphoreType.DMA,
            pltpu.SemaphoreType.DMA,
            pltpu.SemaphoreType.DMA,
            pltpu.SemaphoreType.DMA,
            pltpu.SemaphoreType.DMA,
            pltpu.SemaphoreType.DMA,
        ],
    )
    sh4, length, cut, lg = run(tbl, es2, ed2)
    sh = sh4.transpose(1, 3, 0, 2).reshape(e, 16)[:, :9]
    return (edge_src, edge_dst, sh, length, cut, lg)
